# baseline (speedup 1.0000x reference)
import jax
import jax.numpy as jnp
from jax import lax
from jax.experimental import pallas as pl
from jax.experimental.pallas import tpu as pltpu

_NUM_SE = 2
_HEADS = 4
_CH = 32
_OUT_DIM = 32
_NT = 4
_HID = 64
_B, _N, _F, _GF = 8, 512, 128, 16
_HC = _HEADS * _CH

_TRN = (((0,), (0,)), ((), ()))
_ROW = (((0,), (1,)), ((), ()))


def _fused_body(x_ref, adj_ref, gf_ref, wse_ref, asrc_ref, adst_ref, wfin_ref,
                bfin_ref, w1_ref, b1_ref, w2_ref, b2_ref, out_ref):
    x = x_ref[0]
    row_ids = lax.broadcasted_iota(jnp.int32, (_N, _N), 0)
    col_ids = lax.broadcasted_iota(jnp.int32, (_N, _N), 1)
    eyeb = jnp.where(row_ids == col_ids, 1.0, 0.0).astype(jnp.bfloat16)
    eyef = eyeb.astype(jnp.float32)
    ones_col = jnp.ones((_N, 1), jnp.float32)
    ones_row = jnp.ones((1, _N), jnp.float32)
    log2e = jnp.float32(1.4426950408889634)

    fin_pre = jnp.zeros((_N, _OUT_DIM), jnp.float32)
    for se in range(_NUM_SE):
        ab = adj_ref[se, 0].astype(jnp.bfloat16)
        at = lax.dot_general(ab, eyeb, _TRN,
                             preferred_element_type=jnp.float32)
        maskf = jnp.minimum(ab.astype(jnp.float32) + at + eyef, 1.0)
        h = jnp.dot(x, wse_ref[se], preferred_element_type=jnp.float32)
        outs = []
        for k in range(_HEADS):
            hk = h[:, k * _CH:(k + 1) * _CH]
            ak = asrc_ref[se, :, k:k + 1] * log2e
            dk = adst_ref[se, :, k:k + 1] * log2e
            es = jnp.dot(hk, ak, preferred_element_type=jnp.float32)
            ed = lax.dot_general(dk, hk, _ROW,
                                 preferred_element_type=jnp.float32)
            ed_max = jnp.max(ed, axis=1, keepdims=True)
            lhs = jnp.concatenate([es, ones_col], axis=1)
            rhs = jnp.concatenate([ones_row, ed], axis=0)
            logit = jnp.dot(lhs, rhs, preferred_element_type=jnp.float32)
            logit = jnp.maximum(logit, 0.2 * logit)
            shift = jnp.maximum(es + ed_max, 0.0)
            p = jnp.exp2(logit - shift) * maskf
            h_aug = jnp.concatenate([hk, ones_col], axis=1)
            agg = jnp.dot(p, h_aug, preferred_element_type=jnp.float32)
            outs.append(agg[:, :_CH] * (1.0 / agg[:, _CH:_CH + 1]))
        out_se = jnp.maximum(jnp.concatenate(outs, axis=1), 0.0)
        fin_pre = fin_pre + jnp.dot(out_se, wfin_ref[se * _HC:(se + 1) * _HC, :],
                                    preferred_element_type=jnp.float32)

    fin = jnp.maximum(fin_pre + bfin_ref[:], 0.0)
    pooled = jnp.max(fin, axis=0, keepdims=True)
    gf = gf_ref[0]
    scs = []
    for t in range(_NT):
        h1 = jnp.dot(pooled, w1_ref[t, :_OUT_DIM, :],
                     preferred_element_type=jnp.float32)
        h1 = h1 + jnp.dot(gf, w1_ref[t, _OUT_DIM:, :],
                          preferred_element_type=jnp.float32)
        h1 = jnp.maximum(h1 + b1_ref[t:t + 1, :], 0.0)
        scs.append(jnp.dot(h1, w2_ref[t], preferred_element_type=jnp.float32)
                   + b2_ref[t:t + 1, :])
    sc = jnp.concatenate(scs, axis=1)
    sc = sc - jnp.max(sc, axis=1, keepdims=True)
    ex = jnp.exp(sc)
    out_ref[0] = ex / jnp.sum(ex, axis=1, keepdims=True)


@jax.jit
def kernel(node_features, adjacency, graph_features, W_se, a_src, a_dst,
           W_fin, b_fin, W_dec1, b_dec1, W_dec2, b_dec2):
    asrc = jnp.transpose(a_src, (0, 2, 1))
    adst = jnp.transpose(a_dst, (0, 2, 1))
    bfin_row = b_fin.reshape(1, _OUT_DIM)
    gf3 = graph_features.reshape(_B, 1, _GF)

    full = lambda shape: pl.BlockSpec(shape, lambda b: (0,) * len(shape))
    out = pl.pallas_call(
        _fused_body,
        grid=(_B,),
        in_specs=[
            pl.BlockSpec((1, _N, _F), lambda b: (b, 0, 0)),
            pl.BlockSpec((_NUM_SE, 1, _N, _N), lambda b: (0, b, 0, 0)),
            pl.BlockSpec((1, 1, _GF), lambda b: (b, 0, 0)),
            full((_NUM_SE, _F, _HC)),
            full((_NUM_SE, _CH, _HEADS)),
            full((_NUM_SE, _CH, _HEADS)),
            full((_NUM_SE * _HC, _OUT_DIM)),
            full((1, _OUT_DIM)),
            full((_NT, _OUT_DIM + _GF, _HID)),
            full((_NT, _HID)),
            full((_NT, _HID, 1)),
            full((_NT, 1)),
        ],
        out_specs=pl.BlockSpec((1, 1, _NT), lambda b: (b, 0, 0)),
        out_shape=jax.ShapeDtypeStruct((_B, 1, _NT), jnp.float32),
        compiler_params=pltpu.CompilerParams(
            dimension_semantics=("parallel",),
        ),
    )(node_features, adjacency, gf3, W_se, asrc, adst, W_fin, bfin_row,
      W_dec1, b_dec1, W_dec2, b_dec2)
    return out.reshape(_B, _NT)


# speedup vs baseline: 1.3551x; 1.3551x over previous
import jax
import jax.numpy as jnp
from jax import lax
from jax.experimental import pallas as pl
from jax.experimental.pallas import tpu as pltpu

_NUM_SE = 2
_HEADS = 4
_CH = 32
_OUT_DIM = 32
_NT = 4
_HID = 64
_B, _N, _F, _GF = 8, 512, 128, 16
_HC = _HEADS * _CH

_TRN = (((0,), (0,)), ((), ()))
_TRL = (((1,), (1,)), ((), ()))


def _fused_body(x_ref, adj_ref, gf_ref, wse_ref, asrc_ref, adst_ref, wfin_ref,
                bfin_ref, w1_ref, b1_ref, w2_ref, b2_ref, out_ref):
    x = x_ref[0]
    row_ids = lax.broadcasted_iota(jnp.int32, (_N, _N), 0)
    col_ids = lax.broadcasted_iota(jnp.int32, (_N, _N), 1)
    eyeb = jnp.where(row_ids == col_ids, 1.0, 0.0).astype(jnp.bfloat16)
    eyef = eyeb.astype(jnp.float32)
    ones_col = jnp.ones((_N, 1), jnp.float32)
    ones_row = jnp.ones((1, _N), jnp.float32)
    log2e = jnp.float32(1.4426950408889634)

    stack = jnp.concatenate([asrc_ref[0], adst_ref[0],
                             asrc_ref[1], adst_ref[1]], axis=0)
    eye32 = jnp.where(lax.broadcasted_iota(jnp.int32, (_CH, _CH), 0)
                      == lax.broadcasted_iota(jnp.int32, (_CH, _CH), 1),
                      1.0, 0.0).astype(jnp.float32)
    tv = lax.dot_general(eye32, stack, _TRL,
                         preferred_element_type=jnp.float32)
    tt = jnp.concatenate([tv] * _HEADS, axis=0)
    blk = jnp.where(lax.broadcasted_iota(jnp.int32, (_HC, _HEADS), 0) // _CH
                    == lax.broadcasted_iota(jnp.int32, (_HC, _HEADS), 1),
                    log2e, 0.0).astype(jnp.float32)
    smats = [tt[:, 0:_HEADS] * blk, tt[:, 2 * _HEADS:3 * _HEADS] * blk]
    dmats = [tt[:, _HEADS:2 * _HEADS] * blk, tt[:, 3 * _HEADS:4 * _HEADS] * blk]

    fin_pre = jnp.zeros((_N, _OUT_DIM), jnp.float32)
    for se in range(_NUM_SE):
        ab = adj_ref[se, 0].astype(jnp.bfloat16)
        at = lax.dot_general(ab, eyeb, _TRN,
                             preferred_element_type=jnp.float32)
        maskf = jnp.minimum(ab.astype(jnp.float32) + at + eyef, 1.0)
        h = jnp.dot(x, wse_ref[se], preferred_element_type=jnp.float32)
        es = jnp.dot(h, smats[se], preferred_element_type=jnp.float32)
        ed = lax.dot_general(dmats[se], h, (((0,), (1,)), ((), ())),
                             preferred_element_type=jnp.float32)
        ed_max = jnp.max(ed, axis=1, keepdims=True)
        outs = []
        for k in range(_HEADS):
            lhs = jnp.concatenate([es[:, k:k + 1], ones_col], axis=1)
            rhs = jnp.concatenate([ones_row, ed[k:k + 1, :]], axis=0)
            logit = jnp.dot(lhs, rhs, preferred_element_type=jnp.float32)
            logit = jnp.maximum(logit, 0.2 * logit)
            shift = jnp.maximum(es[:, k:k + 1] + ed_max[k:k + 1, :], 0.0)
            p = jnp.exp2(logit - shift) * maskf
            h_aug = jnp.concatenate(
                [h[:, k * _CH:(k + 1) * _CH], ones_col], axis=1)
            agg = jnp.dot(p, h_aug, preferred_element_type=jnp.float32)
            outs.append(agg[:, :_CH] * (1.0 / agg[:, _CH:_CH + 1]))
        out_se = jnp.maximum(jnp.concatenate(outs, axis=1), 0.0)
        fin_pre = fin_pre + jnp.dot(out_se, wfin_ref[se * _HC:(se + 1) * _HC, :],
                                    preferred_element_type=jnp.float32)

    fin = jnp.maximum(fin_pre + bfin_ref[:], 0.0)
    pooled = jnp.max(fin, axis=0, keepdims=True)
    gf = gf_ref[0]

    w1cat = jnp.concatenate([w1_ref[t] for t in range(_NT)], axis=1)
    b1row = jnp.concatenate([b1_ref[t:t + 1, :] for t in range(_NT)],
                            axis=1)
    w2col = jnp.concatenate([w2_ref[t] for t in range(_NT)], axis=0)
    blk4 = jnp.where(lax.broadcasted_iota(jnp.int32, (_NT * _HID, _NT), 0) // _HID
                     == lax.broadcasted_iota(jnp.int32, (_NT * _HID, _NT), 1),
                     1.0, 0.0).astype(jnp.float32)
    w2bd = w2col * blk4
    b2row = lax.dot_general(b2_ref[:], eyef[:_NT, :_NT], _TRN,
                            preferred_element_type=jnp.float32)
    h1 = jnp.dot(pooled, w1cat[:_OUT_DIM, :], preferred_element_type=jnp.float32)
    h1 = h1 + jnp.dot(gf, w1cat[_OUT_DIM:, :], preferred_element_type=jnp.float32)
    h1 = jnp.maximum(h1 + b1row, 0.0)
    sc = jnp.dot(h1, w2bd, preferred_element_type=jnp.float32) + b2row
    sc = sc - jnp.max(sc, axis=1, keepdims=True)
    ex = jnp.exp(sc)
    out_ref[0] = ex / jnp.sum(ex, axis=1, keepdims=True)


@jax.jit
def kernel(node_features, adjacency, graph_features, W_se, a_src, a_dst,
           W_fin, b_fin, W_dec1, b_dec1, W_dec2, b_dec2):
    bfin_row = b_fin.reshape(1, _OUT_DIM)
    gf3 = graph_features.reshape(_B, 1, _GF)

    full = lambda shape: pl.BlockSpec(shape, lambda b: (0,) * len(shape))
    out = pl.pallas_call(
        _fused_body,
        grid=(_B,),
        in_specs=[
            pl.BlockSpec((1, _N, _F), lambda b: (b, 0, 0)),
            pl.BlockSpec((_NUM_SE, 1, _N, _N), lambda b: (0, b, 0, 0)),
            pl.BlockSpec((1, 1, _GF), lambda b: (b, 0, 0)),
            full((_NUM_SE, _F, _HC)),
            full((_NUM_SE, _HEADS, _CH)),
            full((_NUM_SE, _HEADS, _CH)),
            full((_NUM_SE * _HC, _OUT_DIM)),
            full((1, _OUT_DIM)),
            full((_NT, _OUT_DIM + _GF, _HID)),
            full((_NT, _HID)),
            full((_NT, _HID, 1)),
            full((_NT, 1)),
        ],
        out_specs=pl.BlockSpec((1, 1, _NT), lambda b: (b, 0, 0)),
        out_shape=jax.ShapeDtypeStruct((_B, 1, _NT), jnp.float32),
        compiler_params=pltpu.CompilerParams(
            dimension_semantics=("parallel",),
        ),
    )(node_features, adjacency, gf3, W_se, a_src, a_dst, W_fin, bfin_row,
      W_dec1, b_dec1, W_dec2, b_dec2)
    return out.reshape(_B, _NT)


# speedup vs baseline: 1.3596x; 1.0034x over previous
import jax
import jax.numpy as jnp
from jax import lax
from jax.experimental import pallas as pl
from jax.experimental.pallas import tpu as pltpu

_NUM_SE = 2
_HEADS = 4
_CH = 32
_OUT_DIM = 32
_NT = 4
_HID = 64
_B, _N, _F, _GF = 8, 512, 128, 16
_HC = _HEADS * _CH

_TRN = (((0,), (0,)), ((), ()))
_TRL = (((1,), (1,)), ((), ()))


def _fused_body(x_ref, adj_ref, gf_ref, wse_ref, asrc_ref, adst_ref, wfin_ref,
                bfin_ref, w1_ref, b1_ref, w2_ref, b2_ref, out_ref):
    b = pl.program_id(0)
    x = x_ref[0]
    row_ids = lax.broadcasted_iota(jnp.int32, (_N, _N), 0)
    col_ids = lax.broadcasted_iota(jnp.int32, (_N, _N), 1)
    eyeb = jnp.where(row_ids == col_ids, 1.0, 0.0).astype(jnp.bfloat16)
    eyef = eyeb.astype(jnp.float32)
    ones_col = jnp.ones((_N, 1), jnp.float32)
    ones_row = jnp.ones((1, _N), jnp.float32)
    log2e = jnp.float32(1.4426950408889634)

    stack = jnp.concatenate([asrc_ref[0], adst_ref[0],
                             asrc_ref[1], adst_ref[1]], axis=0)
    eye32 = jnp.where(lax.broadcasted_iota(jnp.int32, (_CH, _CH), 0)
                      == lax.broadcasted_iota(jnp.int32, (_CH, _CH), 1),
                      1.0, 0.0).astype(jnp.float32)
    tv = lax.dot_general(eye32, stack, _TRL,
                         preferred_element_type=jnp.float32)
    tt = jnp.concatenate([tv] * _HEADS, axis=0)
    blk = jnp.where(lax.broadcasted_iota(jnp.int32, (_HC, _HEADS), 0) // _CH
                    == lax.broadcasted_iota(jnp.int32, (_HC, _HEADS), 1),
                    log2e, 0.0).astype(jnp.float32)
    smats = [tt[:, 0:_HEADS] * blk, tt[:, 2 * _HEADS:3 * _HEADS] * blk]
    dmats = [tt[:, _HEADS:2 * _HEADS] * blk, tt[:, 3 * _HEADS:4 * _HEADS] * blk]

    fin_pre = jnp.zeros((_N, _OUT_DIM), jnp.float32)
    for se in range(_NUM_SE):
        ab = adj_ref[se, 0].astype(jnp.bfloat16)
        at = lax.dot_general(ab, eyeb, _TRN,
                             preferred_element_type=jnp.float32)
        maskf = jnp.minimum(ab.astype(jnp.float32) + at + eyef, 1.0)
        h = jnp.dot(x, wse_ref[se], preferred_element_type=jnp.float32)
        es = jnp.dot(h, smats[se], preferred_element_type=jnp.float32)
        ed = lax.dot_general(dmats[se], h, (((0,), (1,)), ((), ())),
                             preferred_element_type=jnp.float32)
        ed_max = jnp.max(ed, axis=1, keepdims=True)
        outs = []
        for k in range(_HEADS):
            lhs = jnp.concatenate([es[:, k:k + 1], ones_col], axis=1)
            rhs = jnp.concatenate([ones_row, ed[k:k + 1, :]], axis=0)
            logit = jnp.dot(lhs, rhs, preferred_element_type=jnp.float32)
            logit = jnp.maximum(logit, 0.2 * logit)
            shift = jnp.maximum(es[:, k:k + 1] + ed_max[k:k + 1, :], 0.0)
            p = jnp.exp2(logit - shift) * maskf
            h_aug = jnp.concatenate(
                [h[:, k * _CH:(k + 1) * _CH], ones_col], axis=1)
            agg = jnp.dot(p, h_aug, preferred_element_type=jnp.float32)
            outs.append(agg[:, :_CH] * (1.0 / agg[:, _CH:_CH + 1]))
        out_se = jnp.maximum(jnp.concatenate(outs, axis=1), 0.0)
        fin_pre = fin_pre + jnp.dot(out_se, wfin_ref[se * _HC:(se + 1) * _HC, :],
                                    preferred_element_type=jnp.float32)

    fin = jnp.maximum(fin_pre + bfin_ref[:], 0.0)
    pooled = jnp.max(fin, axis=0, keepdims=True)
    gf = gf_ref[pl.ds(b, 1), :]

    w1cat = jnp.concatenate([w1_ref[t] for t in range(_NT)], axis=1)
    b1row = jnp.concatenate([b1_ref[t:t + 1, :] for t in range(_NT)],
                            axis=1)
    w2col = jnp.concatenate([w2_ref[t] for t in range(_NT)], axis=0)
    blk4 = jnp.where(lax.broadcasted_iota(jnp.int32, (_NT * _HID, _NT), 0) // _HID
                     == lax.broadcasted_iota(jnp.int32, (_NT * _HID, _NT), 1),
                     1.0, 0.0).astype(jnp.float32)
    w2bd = w2col * blk4
    b2row = lax.dot_general(b2_ref[:], eyef[:_NT, :_NT], _TRN,
                            preferred_element_type=jnp.float32)
    h1 = jnp.dot(pooled, w1cat[:_OUT_DIM, :], preferred_element_type=jnp.float32)
    h1 = h1 + jnp.dot(gf, w1cat[_OUT_DIM:, :], preferred_element_type=jnp.float32)
    h1 = jnp.maximum(h1 + b1row, 0.0)
    sc = jnp.dot(h1, w2bd, preferred_element_type=jnp.float32) + b2row
    sc = sc - jnp.max(sc, axis=1, keepdims=True)
    ex = jnp.exp(sc)
    out_ref[pl.ds(b, 1), :] = ex / jnp.sum(ex, axis=1, keepdims=True)


@jax.jit
def kernel(node_features, adjacency, graph_features, W_se, a_src, a_dst,
           W_fin, b_fin, W_dec1, b_dec1, W_dec2, b_dec2):
    full = lambda shape: pl.BlockSpec(shape, lambda b: (0,) * len(shape))
    out = pl.pallas_call(
        _fused_body,
        grid=(_B,),
        in_specs=[
            pl.BlockSpec((1, _N, _F), lambda b: (b, 0, 0)),
            pl.BlockSpec((_NUM_SE, 1, _N, _N), lambda b: (0, b, 0, 0)),
            full((_B, _GF)),
            full((_NUM_SE, _F, _HC)),
            full((_NUM_SE, _HEADS, _CH)),
            full((_NUM_SE, _HEADS, _CH)),
            full((_NUM_SE * _HC, _OUT_DIM)),
            full((_OUT_DIM,)),
            full((_NT, _OUT_DIM + _GF, _HID)),
            full((_NT, _HID)),
            full((_NT, _HID, 1)),
            full((_NT, 1)),
        ],
        out_specs=full((_B, _NT)),
        out_shape=jax.ShapeDtypeStruct((_B, _NT), jnp.float32),
        compiler_params=pltpu.CompilerParams(
            dimension_semantics=("arbitrary",),
        ),
    )(node_features, adjacency, graph_features, W_se, a_src, a_dst, W_fin,
      b_fin, W_dec1, b_dec1, W_dec2, b_dec2)
    return out
